# baseline (speedup 1.0000x reference)
import functools

import numpy as np
import jax
import jax.numpy as jnp
from jax import lax
from jax.experimental import pallas as pl
from jax.experimental.pallas import tpu as pltpu
from jax.experimental.pallas import tpu_sc as plsc

_B, _L, _V, _D = 4096, 200, 100000, 64
_NC, _NS, _LANES = 2, 16, 16
_NW = _NC * _NS
_BPW = _B // _NW
_NTI = _D // 8
_TILE = 8 * 128
_LSTRIDE = _NTI * (_B // 128) * _TILE
_TISTRIDE = (_B // 128) * _TILE


def _pos_encoding():
    position = np.arange(_L, dtype=np.float32)[:, None]
    div_term = np.exp(
        np.arange(0, _D, 2, dtype=np.float32) * (-np.log(10000.0) / _D)
    ).astype(np.float32)
    pe = np.zeros((_L, _D), np.float32)
    pe[:, 0::2] = np.sin(position * div_term)
    pe[:, 1::2] = np.cos(position * div_term)
    return jnp.asarray(pe)


def _body(pos_hbm, tokens_hbm, table_hbm, out_hbm,
          pos_v, idx_raw, idx_t, g0, g1, o0, o1, gsem0, gsem1, osem0, osem1):
    wid = lax.axis_index("s") * _NC + lax.axis_index("c")
    wb = wid * _BPW
    pltpu.sync_copy(tokens_hbm.at[pl.ds(wb, _BPW)], idx_raw)
    pltpu.sync_copy(pos_hbm, pos_v)

    k16 = lax.iota(jnp.int32, 16)
    tail_mask = k16 >= 8
    addr_c = [
        (2 * j) * 1024 + (k16 >> 3) * 1024 + (k16 & 7) * 128
        for j in range(_D // _LANES)
    ]

    def tr_row(r, c):
        for cj in range(12):
            vals = idx_raw[r, pl.ds(cj * 16, 16)]
            plsc.store_scatter(idx_t, [(k16 + cj * 16) * _BPW + r], vals)
        vals = idx_raw[r, pl.ds(184, 16)]
        plsc.store_scatter(
            idx_t, [(k16 + 184) * _BPW + r], vals, mask=tail_mask
        )
        return c

    lax.fori_loop(0, _BPW, tr_row, 0)

    g = (g0, g1)
    o = (o0, o1)
    gsem = (gsem0, gsem1)
    osem = (osem0, osem1)

    def fire_gather(l, s):
        pltpu.async_copy(
            table_hbm.at[idx_t.at[pl.ds(l * _BPW, _BPW)]], g[s], gsem[s]
        )

    def wait_gather(s):
        pltpu.make_async_copy(
            table_hbm.at[idx_t.at[pl.ds(0, _BPW)]], g[s], gsem[s]
        ).wait()

    def fire_out(l, s):
        for ti in range(_NTI):
            pltpu.async_copy(
                o[s].at[pl.ds(ti * _TILE, _TILE)],
                out_hbm.at[pl.ds(l * _LSTRIDE + ti * _TISTRIDE + wid * _TILE,
                                 _TILE)],
                osem[s],
            )

    def wait_out(s):
        for ti in range(_NTI):
            pltpu.make_async_copy(
                o[s].at[pl.ds(ti * _TILE, _TILE)],
                out_hbm.at[pl.ds(ti * _TILE, _TILE)],
                osem[s],
            ).wait()

    fire_gather(0, 0)

    def step(lo, carry):
        for s in range(2):
            l = 2 * lo + s
            n = 1 - s

            @pl.when(l < _L - 1)
            def _():
                fire_gather(l + 1, n)

            wait_gather(s)

            @pl.when(l > 1)
            def _():
                wait_out(s)

            g_v = g[s]
            o_v = o[s]
            prow = [pos_v[l, pl.ds(j * 16, 16)] for j in range(_D // _LANES)]

            def add_row(r, c):
                for j in range(_D // _LANES):
                    vals = g_v[r, pl.ds(j * 16, 16)] + prow[j]
                    plsc.store_scatter(o_v, [addr_c[j] + r], vals)
                return c

            lax.fori_loop(0, _BPW, add_row, 0)
            fire_out(l, s)
        return carry

    lax.fori_loop(0, _L // 2, step, 0)
    wait_out(0)
    wait_out(1)


@functools.partial(jax.jit, static_argnums=())
def kernel(tokens, table):
    pos = _pos_encoding()
    mesh = plsc.VectorSubcoreMesh(core_axis_name="c", subcore_axis_name="s")
    run = pl.kernel(
        _body,
        out_type=jax.ShapeDtypeStruct((_B * _L * _D,), jnp.float32),
        mesh=mesh,
        scratch_types=[
            pltpu.VMEM((_L, _D), jnp.float32),
            pltpu.VMEM((_BPW, _L), jnp.int32),
            pltpu.VMEM((_BPW * _L,), jnp.int32),
            pltpu.VMEM((_BPW, _D), jnp.float32),
            pltpu.VMEM((_BPW, _D), jnp.float32),
            pltpu.VMEM((_NTI * _TILE,), jnp.float32),
            pltpu.VMEM((_NTI * _TILE,), jnp.float32),
            pltpu.SemaphoreType.DMA,
            pltpu.SemaphoreType.DMA,
            pltpu.SemaphoreType.DMA,
            pltpu.SemaphoreType.DMA,
        ],
        compiler_params=pltpu.CompilerParams(
            use_tc_tiling_on_sc=False, needs_layout_passes=False
        ),
    )
    out_flat = run(pos, tokens, table)
    x = out_flat.reshape(_L, _NTI, _B // 128, 8, 128)
    return x.transpose(2, 4, 0, 1, 3).reshape(_B, _L, _D)


# speedup vs baseline: 1.0187x; 1.0187x over previous
import functools

import numpy as np
import jax
import jax.numpy as jnp
from jax import lax
from jax.experimental import pallas as pl
from jax.experimental.pallas import tpu as pltpu
from jax.experimental.pallas import tpu_sc as plsc

_B, _L, _V, _D = 4096, 200, 100000, 64
_NC, _NS, _LANES = 2, 16, 16
_NW = _NC * _NS
_BPW = _B // _NW
_NTI = _D // 8
_TILE = 8 * 128
_LSTRIDE = _NTI * (_B // 128) * _TILE
_TISTRIDE = (_B // 128) * _TILE
_RUNROLL = 8


def _pos_encoding():
    position = np.arange(_L, dtype=np.float32)[:, None]
    div_term = np.exp(
        np.arange(0, _D, 2, dtype=np.float32) * (-np.log(10000.0) / _D)
    ).astype(np.float32)
    pe = np.zeros((_L, _D), np.float32)
    pe[:, 0::2] = np.sin(position * div_term)
    pe[:, 1::2] = np.cos(position * div_term)
    return jnp.asarray(pe)


def _body(pos_hbm, tokens_hbm, table_hbm, out_hbm,
          pos_v, idx_raw, idx_t, g0, g1, o0, o1, gsem0, gsem1, osem0, osem1):
    wid = lax.axis_index("s") * _NC + lax.axis_index("c")
    wb = wid * _BPW
    pltpu.sync_copy(tokens_hbm.at[pl.ds(wb, _BPW)], idx_raw)
    pltpu.sync_copy(pos_hbm, pos_v)

    k16 = lax.iota(jnp.int32, 16)
    tail_mask = k16 >= 8
    addr_c = [
        (2 * j) * 1024 + (k16 >> 3) * 1024 + (k16 & 7) * 128
        for j in range(_D // _LANES)
    ]

    def tr_row(r, c):
        for cj in range(12):
            vals = idx_raw[r, pl.ds(cj * 16, 16)]
            plsc.store_scatter(idx_t, [(k16 + cj * 16) * _BPW + r], vals)
        vals = idx_raw[r, pl.ds(184, 16)]
        plsc.store_scatter(
            idx_t, [(k16 + 184) * _BPW + r], vals, mask=tail_mask
        )
        return c

    lax.fori_loop(0, _BPW, tr_row, 0)

    g = (g0, g1)
    o = (o0, o1)
    gsem = (gsem0, gsem1)
    osem = (osem0, osem1)

    def fire_gather(l, s):
        pltpu.async_copy(
            table_hbm.at[idx_t.at[pl.ds(l * _BPW, _BPW)]], g[s], gsem[s]
        )

    def wait_gather(s):
        pltpu.make_async_copy(
            table_hbm.at[idx_t.at[pl.ds(0, _BPW)]], g[s], gsem[s]
        ).wait()

    def fire_out(l, s):
        for ti in range(_NTI):
            pltpu.async_copy(
                o[s].at[pl.ds(ti * _TILE, _TILE)],
                out_hbm.at[pl.ds(l * _LSTRIDE + ti * _TISTRIDE + wid * _TILE,
                                 _TILE)],
                osem[s],
            )

    def wait_out(s):
        for ti in range(_NTI):
            pltpu.make_async_copy(
                o[s].at[pl.ds(ti * _TILE, _TILE)],
                out_hbm.at[pl.ds(ti * _TILE, _TILE)],
                osem[s],
            ).wait()

    fire_gather(0, 0)

    def step(lo, carry):
        for s in range(2):
            l = 2 * lo + s
            n = 1 - s

            @pl.when(l < _L - 1)
            def _():
                fire_gather(l + 1, n)

            wait_gather(s)

            @pl.when(l > 1)
            def _():
                wait_out(s)

            g_v = g[s]
            o_v = o[s]
            prow = [pos_v[l, pl.ds(j * 16, 16)] for j in range(_D // _LANES)]

            def add_row(ro, c):
                r0 = ro * _RUNROLL
                for rr in range(_RUNROLL):
                    for j in range(_D // _LANES):
                        vals = g_v[r0 + rr, pl.ds(j * 16, 16)] + prow[j]
                        plsc.store_scatter(o_v, [addr_c[j] + (r0 + rr)], vals)
                return c

            lax.fori_loop(0, _BPW // _RUNROLL, add_row, 0)
            fire_out(l, s)
        return carry

    lax.fori_loop(0, _L // 2, step, 0)
    wait_out(0)
    wait_out(1)


@functools.partial(jax.jit, static_argnums=())
def kernel(tokens, table):
    pos = _pos_encoding()
    mesh = plsc.VectorSubcoreMesh(core_axis_name="c", subcore_axis_name="s")
    run = pl.kernel(
        _body,
        out_type=jax.ShapeDtypeStruct((_B * _L * _D,), jnp.float32),
        mesh=mesh,
        scratch_types=[
            pltpu.VMEM((_L, _D), jnp.float32),
            pltpu.VMEM((_BPW, _L), jnp.int32),
            pltpu.VMEM((_BPW * _L,), jnp.int32),
            pltpu.VMEM((_BPW, _D), jnp.float32),
            pltpu.VMEM((_BPW, _D), jnp.float32),
            pltpu.VMEM((_NTI * _TILE,), jnp.float32),
            pltpu.VMEM((_NTI * _TILE,), jnp.float32),
            pltpu.SemaphoreType.DMA,
            pltpu.SemaphoreType.DMA,
            pltpu.SemaphoreType.DMA,
            pltpu.SemaphoreType.DMA,
        ],
        compiler_params=pltpu.CompilerParams(
            use_tc_tiling_on_sc=False, needs_layout_passes=False
        ),
    )
    out_flat = run(pos, tokens, table)
    x = out_flat.reshape(_L, _NTI, _B // 128, 8, 128)
    return x.transpose(2, 4, 0, 1, 3).reshape(_B, _L, _D)


# speedup vs baseline: 2.9421x; 2.8881x over previous
import functools

import numpy as np
import jax
import jax.numpy as jnp
from jax import lax
from jax.experimental import pallas as pl
from jax.experimental.pallas import tpu as pltpu
from jax.experimental.pallas import tpu_sc as plsc

_B, _L, _V, _D = 4096, 200, 100000, 64
_DP = 128
_NC, _NS, _LANES = 2, 16, 16
_NW = _NC * _NS
_BPW = _B // _NW
_LA, _LB = 104, 96
_RUNROLL = 8


def _pos_encoding():
    position = np.arange(_L, dtype=np.float32)[:, None]
    div_term = np.exp(
        np.arange(0, _D, 2, dtype=np.float32) * (-np.log(10000.0) / _D)
    ).astype(np.float32)
    pe = np.zeros((_L, _D), np.float32)
    pe[:, 0::2] = np.sin(position * div_term)
    pe[:, 1::2] = np.cos(position * div_term)
    return jnp.asarray(pe)


def _body(pos_hbm, tokens_hbm, table_hbm, out_hbm,
          pos_v, idx_v, rows0, rows1, o0, o1, gsem0, gsem1, osem0, osem1):
    wid = lax.axis_index("s") * _NC + lax.axis_index("c")
    base = wid * _BPW
    pltpu.sync_copy(tokens_hbm.at[pl.ds(base, _BPW)], idx_v)
    pltpu.sync_copy(pos_hbm, pos_v)

    rows = (rows0, rows1)
    outs = (o0, o1)
    gsem = (gsem0, gsem1)
    osem = (osem0, osem1)

    def fire_gather(i, s):
        pltpu.async_copy(
            table_hbm.at[idx_v.at[i, pl.ds(0, _LA)]],
            rows[s].at[pl.ds(0, _LA)], gsem[s]
        )
        pltpu.async_copy(
            table_hbm.at[idx_v.at[i, pl.ds(_LA, _LB)]],
            rows[s].at[pl.ds(_LA, _LB)], gsem[s]
        )

    def wait_gather(s):
        pltpu.make_async_copy(
            table_hbm.at[idx_v.at[0, pl.ds(0, _LA)]],
            rows[s].at[pl.ds(0, _LA)], gsem[s]
        ).wait()
        pltpu.make_async_copy(
            table_hbm.at[idx_v.at[0, pl.ds(_LA, _LB)]],
            rows[s].at[pl.ds(_LA, _LB)], gsem[s]
        ).wait()

    def fire_out(i, s):
        pltpu.async_copy(
            outs[s],
            out_hbm.at[pl.ds((base + i) * _L, _L), pl.ds(0, _D)],
            osem[s],
        )

    def wait_out(s):
        pltpu.make_async_copy(
            outs[s],
            out_hbm.at[pl.ds(base * _L, _L), pl.ds(0, _D)],
            osem[s],
        ).wait()

    fire_gather(0, 0)

    def step(ko, carry):
        for s in range(2):
            i = 2 * ko + s
            n = 1 - s

            @pl.when(i < _BPW - 1)
            def _():
                fire_gather(i + 1, n)

            wait_gather(s)

            r_v = rows[s]
            w_v = outs[s]

            @pl.when(i > 1)
            def _():
                wait_out(s)

            def add_rows(ro, c):
                r0 = ro * _RUNROLL
                for rr in range(_RUNROLL):
                    for j in range(_D // _LANES):
                        sl = pl.ds(j * _LANES, _LANES)
                        w_v[r0 + rr, sl] = r_v[r0 + rr, sl] + pos_v[r0 + rr, sl]
                return c

            lax.fori_loop(0, _L // _RUNROLL, add_rows, 0)
            fire_out(i, s)
        return carry

    lax.fori_loop(0, _BPW // 2, step, 0)
    wait_out(0)
    wait_out(1)


@functools.partial(jax.jit, static_argnums=())
def kernel(tokens, table):
    pos = _pos_encoding()
    mesh = plsc.VectorSubcoreMesh(core_axis_name="c", subcore_axis_name="s")
    run = pl.kernel(
        _body,
        out_type=jax.ShapeDtypeStruct((_B * _L, _DP), jnp.float32),
        mesh=mesh,
        scratch_types=[
            pltpu.VMEM((_L, _D), jnp.float32),
            pltpu.VMEM((_BPW, _L), jnp.int32),
            pltpu.VMEM((_L, _D), jnp.float32),
            pltpu.VMEM((_L, _D), jnp.float32),
            pltpu.VMEM((_L, _D), jnp.float32),
            pltpu.VMEM((_L, _D), jnp.float32),
            pltpu.SemaphoreType.DMA,
            pltpu.SemaphoreType.DMA,
            pltpu.SemaphoreType.DMA,
            pltpu.SemaphoreType.DMA,
        ],
        compiler_params=pltpu.CompilerParams(use_tc_tiling_on_sc=False),
    )
    out_p = run(pos, tokens, table)
    return out_p[:, :_D].reshape(_B, _L, _D)
